# baseline (speedup 1.0000x reference)
import functools

import jax
import jax.numpy as jnp
from jax import lax
from jax.experimental import pallas as pl
from jax.experimental.pallas import tpu as pltpu
from jax.experimental.pallas import tpu_sc as plsc

B = 4096
D = 64
L = 16
NC, NS = 2, 16
NW = NC * NS
RPW = B // NW
GROUPS = RPW // L
KCH = D // L

_DN = lax.GatherDimensionNumbers(
    offset_dims=(), collapsed_slice_dims=(0,), start_index_map=(0,))


def _lane_gather(x, idx):
    return lax.gather(x, idx[:, None], dimension_numbers=_DN,
                      slice_sizes=(1,),
                      mode=lax.GatherScatterMode.PROMISE_IN_BOUNDS)


def _gmf_body(uidx_hbm, iidx_hbm, utab_hbm, itab_hbm, wb_hbm, out_hbm,
              uidx_v, iidx_v, urows_v, irows_v, wb_v, res_v, sem_u, sem_i):
    wid = lax.axis_index("s") * NC + lax.axis_index("c")
    base = wid * RPW

    pltpu.sync_copy(wb_hbm, wb_v)
    pltpu.sync_copy(uidx_hbm.at[pl.ds(base, RPW)], uidx_v)
    pltpu.sync_copy(iidx_hbm.at[pl.ds(base, RPW)], iidx_v)
    cu = pltpu.async_copy(utab_hbm.at[uidx_v], urows_v, sem_u)
    ci = pltpu.async_copy(itab_hbm.at[iidx_v], irows_v, sem_i)
    cu.wait()
    ci.wait()

    w_vecs = [wb_v[pl.ds(k * L, L)] for k in range(KCH)]
    b_vec = wb_v[pl.ds(D, L)]
    lane = lax.iota(jnp.int32, L)

    def group(g, carry):
        acc = jnp.zeros((L,), jnp.float32)
        for r in range(L):
            row = g * L + r
            p = jnp.zeros((L,), jnp.float32)
            for k in range(KCH):
                u = urows_v[row, pl.ds(k * L, L)]
                v = irows_v[row, pl.ds(k * L, L)]
                p = p + u * v * w_vecs[k]
            for s in (8, 4, 2, 1):
                p = p + _lane_gather(p, lane ^ s)
            acc = jnp.where(lane == r, p, acc)
        z = acc + b_vec
        res_v[pl.ds(g * L, L)] = 1.0 / (1.0 + jnp.exp(-z))
        return carry

    lax.fori_loop(0, GROUPS, group, 0)
    pltpu.sync_copy(res_v, out_hbm.at[pl.ds(base, RPW)])


_gmf = functools.partial(
    pl.kernel,
    mesh=plsc.VectorSubcoreMesh(core_axis_name="c", subcore_axis_name="s"),
    compiler_params=pltpu.CompilerParams(use_tc_tiling_on_sc=False),
    out_type=jax.ShapeDtypeStruct((B,), jnp.float32),
    scratch_types=[
        pltpu.VMEM((RPW,), jnp.int32),
        pltpu.VMEM((RPW,), jnp.int32),
        pltpu.VMEM((RPW, D), jnp.float32),
        pltpu.VMEM((RPW, D), jnp.float32),
        pltpu.VMEM((D + L,), jnp.float32),
        pltpu.VMEM((RPW,), jnp.float32),
        pltpu.SemaphoreType.DMA,
        pltpu.SemaphoreType.DMA,
    ],
)(_gmf_body)


def kernel(user_indices, item_indices, user_table, item_table, W, b):
    ui = user_indices.astype(jnp.int32)
    ii = item_indices.astype(jnp.int32)
    wb = jnp.concatenate(
        [W.reshape(D), jnp.broadcast_to(b.astype(jnp.float32), (L,))])
    out = _gmf(ui, ii, user_table, item_table, wb)
    return out.reshape(B, 1)


# speedup vs baseline: 1.5775x; 1.5775x over previous
import functools

import jax
import jax.numpy as jnp
from jax import lax
from jax.experimental import pallas as pl
from jax.experimental.pallas import tpu as pltpu
from jax.experimental.pallas import tpu_sc as plsc

B = 4096
D = 64
L = 16
NC, NS = 2, 16
NW = NC * NS
RPW = B // NW
CH = 16
NCHUNK = RPW // CH
KCH = D // L

_DN = lax.GatherDimensionNumbers(
    offset_dims=(), collapsed_slice_dims=(0,), start_index_map=(0,))


def _lane_gather(x, idx):
    return lax.gather(x, idx[:, None], dimension_numbers=_DN,
                      slice_sizes=(1,),
                      mode=lax.GatherScatterMode.PROMISE_IN_BOUNDS)


def _gmf_body(uidx_hbm, iidx_hbm, utab_hbm, itab_hbm, wb_hbm, out_hbm,
              uidx_v, iidx_v, ubufs, ibufs, wb_v, res_v, usems, isems):
    wid = lax.axis_index("s") * NC + lax.axis_index("c")
    base = wid * RPW

    pltpu.sync_copy(wb_hbm, wb_v)
    pltpu.sync_copy(uidx_hbm.at[pl.ds(base, RPW)], uidx_v)
    pltpu.sync_copy(iidx_hbm.at[pl.ds(base, RPW)], iidx_v)

    def fire(g):
        slot = g % 2
        ur = uidx_v[pl.ds(g * CH, CH)]
        ir = iidx_v[pl.ds(g * CH, CH)]
        cps = []
        for r in range(CH):
            cps.append(pltpu.async_copy(
                utab_hbm.at[ur[r]], ubufs[slot].at[r], usems[slot]))
            cps.append(pltpu.async_copy(
                itab_hbm.at[ir[r]], ibufs[slot].at[r], isems[slot]))
        return cps

    w_vecs = [wb_v[pl.ds(k * L, L)] for k in range(KCH)]
    b_vec = wb_v[pl.ds(D, L)]
    lane = lax.iota(jnp.int32, L)

    pend = [fire(0), fire(1)]
    for g in range(NCHUNK):
        slot = g % 2
        for cp in pend[g]:
            cp.wait()
        acc = jnp.zeros((L,), jnp.float32)
        for r in range(CH):
            p = jnp.zeros((L,), jnp.float32)
            for k in range(KCH):
                u = ubufs[slot][r, pl.ds(k * L, L)]
                v = ibufs[slot][r, pl.ds(k * L, L)]
                p = p + u * v * w_vecs[k]
            for s in (8, 4, 2, 1):
                p = p + _lane_gather(p, lane ^ s)
            acc = jnp.where(lane == r, p, acc)
        if g + 2 < NCHUNK:
            pend.append(fire(g + 2))
        z = acc + b_vec
        res_v[pl.ds(g * CH, CH)] = 1.0 / (1.0 + jnp.exp(-z))

    pltpu.sync_copy(res_v, out_hbm.at[pl.ds(base, RPW)])


_gmf = functools.partial(
    pl.kernel,
    mesh=plsc.VectorSubcoreMesh(core_axis_name="c", subcore_axis_name="s"),
    out_type=jax.ShapeDtypeStruct((B,), jnp.float32),
    scratch_types=[
        pltpu.VMEM((RPW,), jnp.int32),
        pltpu.VMEM((RPW,), jnp.int32),
        [pltpu.VMEM((CH, D), jnp.float32) for _ in range(2)],
        [pltpu.VMEM((CH, D), jnp.float32) for _ in range(2)],
        pltpu.VMEM((D + L,), jnp.float32),
        pltpu.VMEM((RPW,), jnp.float32),
        [pltpu.SemaphoreType.DMA for _ in range(2)],
        [pltpu.SemaphoreType.DMA for _ in range(2)],
    ],
)(_gmf_body)


def kernel(user_indices, item_indices, user_table, item_table, W, b):
    ui = user_indices.astype(jnp.int32)
    ii = item_indices.astype(jnp.int32)
    wb = jnp.concatenate(
        [W.reshape(D), jnp.broadcast_to(b.astype(jnp.float32), (L,))])
    out = _gmf(ui, ii, user_table, item_table, wb)
    return out.reshape(B, 1)


# speedup vs baseline: 8.9015x; 5.6426x over previous
import functools

import jax
import jax.numpy as jnp
from jax import lax
from jax.experimental import pallas as pl
from jax.experimental.pallas import tpu as pltpu
from jax.experimental.pallas import tpu_sc as plsc

B = 4096
D = 64
L = 16
NC, NS = 2, 16
NW = NC * NS
RPW = B // NW
NG = RPW // L
NBUF = 4
KCH = D // L

_DN = lax.GatherDimensionNumbers(
    offset_dims=(), collapsed_slice_dims=(0,), start_index_map=(0,))


def _lane_gather(x, idx):
    return lax.gather(x, idx[:, None], dimension_numbers=_DN,
                      slice_sizes=(1,),
                      mode=lax.GatherScatterMode.PROMISE_IN_BOUNDS)


def _gmf_body(uidx_hbm, iidx_hbm, utab_hbm, itab_hbm, wb_hbm, out_hbm,
              uidx_v, iidx_v, ubufs, ibufs, wb_v, res_v, usems, isems):
    wid = lax.axis_index("s") * NC + lax.axis_index("c")
    base = wid * RPW

    pltpu.sync_copy(wb_hbm, wb_v)
    pltpu.sync_copy(uidx_hbm.at[pl.ds(base, RPW)], uidx_v)
    pltpu.sync_copy(iidx_hbm.at[pl.ds(base, RPW)], iidx_v)

    lane = lax.iota(jnp.int32, L)
    dvecs = [lane + k * L for k in range(KCH)]
    w_vecs = [wb_v[pl.ds(k * L, L)] for k in range(KCH)]
    b_vec = wb_v[pl.ds(D, L)]

    ucol, icol, ucin, icin = [], [], [], []
    for g in range(NG):
        uv = uidx_v[pl.ds(g * L, L)]
        iv = iidx_v[pl.ds(g * L, L)]
        ucol.append(lax.shift_left(lax.shift_right_logical(uv, 7), 7))
        icol.append(lax.shift_left(lax.shift_right_logical(iv, 7), 7))
        ucin.append(uv & 127)
        icin.append(iv & 127)

    def fire(e):
        slot = e % NBUF
        g, r = e // L, e % L
        cu = pl.multiple_of(ucol[g][r], 128)
        ci = pl.multiple_of(icol[g][r], 128)
        du = pltpu.async_copy(
            utab_hbm.at[:, pl.ds(cu, 128)], ubufs[slot], usems[slot])
        di = pltpu.async_copy(
            itab_hbm.at[:, pl.ds(ci, 128)], ibufs[slot], isems[slot])
        return du, di

    pend = {e: fire(e) for e in range(NBUF)}
    for e in range(RPW):
        g, r = e // L, e % L
        slot = e % NBUF
        du, di = pend.pop(e)
        du.wait()
        di.wait()
        cu = jnp.broadcast_to(ucin[g][r], (L,))
        ci = jnp.broadcast_to(icin[g][r], (L,))
        p = jnp.zeros((L,), jnp.float32)
        for k in range(KCH):
            u = plsc.load_gather(ubufs[slot], [dvecs[k], cu])
            v = plsc.load_gather(ibufs[slot], [dvecs[k], ci])
            p = p + u * v * w_vecs[k]
        if e + NBUF < RPW:
            pend[e + NBUF] = fire(e + NBUF)
        for s in (8, 4, 2, 1):
            p = p + _lane_gather(p, lane ^ s)
        if r == 0:
            acc = p
        else:
            acc = jnp.where(lane == r, p, acc)
        if r == L - 1:
            z = acc + b_vec
            res_v[pl.ds(g * L, L)] = 1.0 / (1.0 + jnp.exp(-z))

    pltpu.sync_copy(res_v, out_hbm.at[pl.ds(base, RPW)])


_gmf = functools.partial(
    pl.kernel,
    mesh=plsc.VectorSubcoreMesh(core_axis_name="c", subcore_axis_name="s"),
    compiler_params=pltpu.CompilerParams(needs_layout_passes=False),
    out_type=jax.ShapeDtypeStruct((B,), jnp.float32),
    scratch_types=[
        pltpu.VMEM((RPW,), jnp.int32),
        pltpu.VMEM((RPW,), jnp.int32),
        [pltpu.VMEM((D, 128), jnp.float32) for _ in range(NBUF)],
        [pltpu.VMEM((D, 128), jnp.float32) for _ in range(NBUF)],
        pltpu.VMEM((D + L,), jnp.float32),
        pltpu.VMEM((RPW,), jnp.float32),
        [pltpu.SemaphoreType.DMA for _ in range(NBUF)],
        [pltpu.SemaphoreType.DMA for _ in range(NBUF)],
    ],
)(_gmf_body)


def kernel(user_indices, item_indices, user_table, item_table, W, b):
    ui = user_indices.astype(jnp.int32)
    ii = item_indices.astype(jnp.int32)
    ut = user_table.T
    it = item_table.T
    wb = jnp.concatenate(
        [W.reshape(D), jnp.broadcast_to(b.astype(jnp.float32), (L,))])
    out = _gmf(ui, ii, ut, it, wb)
    return out.reshape(B, 1)


# speedup vs baseline: 9.0057x; 1.0117x over previous
import functools

import jax
import jax.numpy as jnp
from jax import lax
from jax.experimental import pallas as pl
from jax.experimental.pallas import tpu as pltpu
from jax.experimental.pallas import tpu_sc as plsc

B = 4096
D = 64
L = 16
NC, NS = 2, 16
NW = NC * NS
RPW = B // NW
NG = RPW // L
NBUF = 6
KCH = D // L

_DN = lax.GatherDimensionNumbers(
    offset_dims=(), collapsed_slice_dims=(0,), start_index_map=(0,))


def _lane_gather(x, idx):
    return lax.gather(x, idx[:, None], dimension_numbers=_DN,
                      slice_sizes=(1,),
                      mode=lax.GatherScatterMode.PROMISE_IN_BOUNDS)


def _gmf_body(uidx_hbm, iidx_hbm, utab_hbm, itab_hbm, wb_hbm, out_hbm,
              uidx_v, iidx_v, ubufs, ibufs, wb_v, res_v, usems, isems):
    wid = lax.axis_index("s") * NC + lax.axis_index("c")
    base = wid * RPW

    pltpu.sync_copy(wb_hbm, wb_v)
    pltpu.sync_copy(uidx_hbm.at[pl.ds(base, RPW)], uidx_v)
    pltpu.sync_copy(iidx_hbm.at[pl.ds(base, RPW)], iidx_v)

    lane = lax.iota(jnp.int32, L)
    dvecs = [lane + k * L for k in range(KCH)]
    w_vecs = [wb_v[pl.ds(k * L, L)] for k in range(KCH)]
    b_vec = wb_v[pl.ds(D, L)]

    ucol, icol, ucin, icin = [], [], [], []
    for g in range(NG):
        uv = uidx_v[pl.ds(g * L, L)]
        iv = iidx_v[pl.ds(g * L, L)]
        ucol.append(lax.shift_left(lax.shift_right_logical(uv, 7), 7))
        icol.append(lax.shift_left(lax.shift_right_logical(iv, 7), 7))
        ucin.append(uv & 127)
        icin.append(iv & 127)

    def fire(e):
        slot = e % NBUF
        g, r = e // L, e % L
        cu = pl.multiple_of(ucol[g][r], 128)
        ci = pl.multiple_of(icol[g][r], 128)
        du = pltpu.async_copy(
            utab_hbm.at[:, pl.ds(cu, 128)], ubufs[slot], usems[slot])
        di = pltpu.async_copy(
            itab_hbm.at[:, pl.ds(ci, 128)], ibufs[slot], isems[slot])
        return du, di

    pend = {e: fire(e) for e in range(NBUF)}
    for e in range(RPW):
        g, r = e // L, e % L
        slot = e % NBUF
        du, di = pend.pop(e)
        du.wait()
        di.wait()
        cu = jnp.broadcast_to(ucin[g][r], (L,))
        ci = jnp.broadcast_to(icin[g][r], (L,))
        p = jnp.zeros((L,), jnp.float32)
        for k in range(KCH):
            u = plsc.load_gather(ubufs[slot], [dvecs[k], cu])
            v = plsc.load_gather(ibufs[slot], [dvecs[k], ci])
            p = p + u * v * w_vecs[k]
        if e + NBUF < RPW:
            pend[e + NBUF] = fire(e + NBUF)
        for s in (8, 4, 2, 1):
            p = p + _lane_gather(p, lane ^ s)
        if r == 0:
            acc = p
        else:
            acc = jnp.where(lane == r, p, acc)
        if r == L - 1:
            z = acc + b_vec
            res_v[pl.ds(g * L, L)] = 1.0 / (1.0 + jnp.exp(-z))

    pltpu.sync_copy(res_v, out_hbm.at[pl.ds(base, RPW)])


_gmf = functools.partial(
    pl.kernel,
    mesh=plsc.VectorSubcoreMesh(core_axis_name="c", subcore_axis_name="s"),
    compiler_params=pltpu.CompilerParams(needs_layout_passes=False),
    out_type=jax.ShapeDtypeStruct((B,), jnp.float32),
    scratch_types=[
        pltpu.VMEM((RPW,), jnp.int32),
        pltpu.VMEM((RPW,), jnp.int32),
        [pltpu.VMEM((D, 128), jnp.float32) for _ in range(NBUF)],
        [pltpu.VMEM((D, 128), jnp.float32) for _ in range(NBUF)],
        pltpu.VMEM((D + L,), jnp.float32),
        pltpu.VMEM((RPW,), jnp.float32),
        [pltpu.SemaphoreType.DMA for _ in range(NBUF)],
        [pltpu.SemaphoreType.DMA for _ in range(NBUF)],
    ],
)(_gmf_body)


def kernel(user_indices, item_indices, user_table, item_table, W, b):
    ui = user_indices.astype(jnp.int32)
    ii = item_indices.astype(jnp.int32)
    ut = user_table.T
    it = item_table.T
    wb = jnp.concatenate(
        [W.reshape(D), jnp.broadcast_to(b.astype(jnp.float32), (L,))])
    out = _gmf(ui, ii, ut, it, wb)
    return out.reshape(B, 1)
